# baseline (speedup 1.0000x reference)
import jax
import jax.numpy as jnp
from jax.experimental import pallas as pl
from jax.experimental.pallas import tpu as pltpu

N_NODES = 10000
D_FEAT = 128
MASK_NUM = 2000

N_CHUNK = 5
CHUNK = N_NODES // N_CHUNK
TRIPS = 25
ROWS_PER_TRIP = CHUNK // TRIPS
IDX_PER_TRIP = MASK_NUM // (N_CHUNK * TRIPS)
N_OUT = 8
OCH = N_NODES // N_OUT


def _attrmask_body(idx_ref, x_ref, ei_ref, out_ref, ei_out_ref, buf_ref,
                   mask_ref, sem_in, sem_out, sem_ei):
    cp_ei = pltpu.make_async_copy(ei_ref, ei_out_ref, sem_ei)
    cp_ei.start()
    cps_in = [
        pltpu.make_async_copy(
            x_ref.at[pl.ds(k * CHUNK, CHUNK), :],
            buf_ref.at[pl.ds(k * CHUNK, CHUNK), :],
            sem_in,
        )
        for k in range(N_CHUNK)
    ]
    for cp in cps_in:
        cp.start()

    mask_ref[...] = jnp.zeros((N_NODES, 1), jnp.float32)
    one = jnp.ones((1, 1), jnp.float32)

    def fused_body(i, acc):
        for j in range(IDX_PER_TRIP):
            mask_ref[pl.ds(idx_ref[i * IDX_PER_TRIP + j], 1), :] = one
        for j in range(ROWS_PER_TRIP // 8):
            acc = acc + buf_ref[pl.ds(i * ROWS_PER_TRIP + j * 8, 8), :]
        return acc

    acc = jnp.zeros((8, D_FEAT), jnp.float32)
    for k in range(N_CHUNK):
        cps_in[k].wait()
        acc = jax.lax.fori_loop(k * TRIPS, (k + 1) * TRIPS, fused_body, acc)
    token = jnp.sum(acc, axis=0, keepdims=True) * (1.0 / N_NODES)

    cps_out = [
        pltpu.make_async_copy(
            buf_ref.at[pl.ds(k * OCH, OCH), :],
            out_ref.at[pl.ds(k * OCH, OCH), :],
            sem_out,
        )
        for k in range(N_OUT)
    ]
    for k in range(N_OUT):
        sl = pl.ds(k * OCH, OCH)
        m = mask_ref[sl, :]
        buf_ref[sl, :] = jnp.where(m > 0.0, token, buf_ref[sl, :])
        cps_out[k].start()
    for k in range(N_OUT):
        cps_out[k].wait()
    cp_ei.wait()


def kernel(x, edge_index, mask_idx):
    idx = mask_idx.astype(jnp.int32)
    x_new, ei_new = pl.pallas_call(
        _attrmask_body,
        out_shape=(
            jax.ShapeDtypeStruct(x.shape, x.dtype),
            jax.ShapeDtypeStruct(edge_index.shape, edge_index.dtype),
        ),
        in_specs=[
            pl.BlockSpec(memory_space=pltpu.SMEM),
            pl.BlockSpec(memory_space=pltpu.HBM),
            pl.BlockSpec(memory_space=pltpu.HBM),
        ],
        out_specs=(
            pl.BlockSpec(memory_space=pltpu.HBM),
            pl.BlockSpec(memory_space=pltpu.HBM),
        ),
        scratch_shapes=[
            pltpu.VMEM((N_NODES, D_FEAT), jnp.float32),
            pltpu.VMEM((N_NODES, 1), jnp.float32),
            pltpu.SemaphoreType.DMA,
            pltpu.SemaphoreType.DMA,
            pltpu.SemaphoreType.DMA,
        ],
    )(idx, x, edge_index)
    return (x_new, ei_new)


# speedup vs baseline: 8.6567x; 8.6567x over previous
import jax
import jax.numpy as jnp
from jax.experimental import pallas as pl
from jax.experimental.pallas import tpu as pltpu

N_NODES = 10000
D_FEAT = 128
MASK_NUM = 2000

N_CHUNK = 5
CHUNK = N_NODES // N_CHUNK
TRIPS = 25
ROWS_PER_TRIP = CHUNK // TRIPS
IDX_PER_TRIP = MASK_NUM // (N_CHUNK * TRIPS)
N_OUT = 8
OCH = N_NODES // N_OUT


def _attrmask_body(idx_ref, x_ref, out_ref, buf_ref, mask_ref, sem_in, sem_out):
    cps_in = [
        pltpu.make_async_copy(
            x_ref.at[pl.ds(k * CHUNK, CHUNK), :],
            buf_ref.at[pl.ds(k * CHUNK, CHUNK), :],
            sem_in,
        )
        for k in range(N_CHUNK)
    ]
    for cp in cps_in:
        cp.start()

    mask_ref[...] = jnp.zeros((N_NODES, 1), jnp.float32)
    one = jnp.ones((1, 1), jnp.float32)

    def fused_body(i, acc):
        for j in range(IDX_PER_TRIP):
            mask_ref[pl.ds(idx_ref[i * IDX_PER_TRIP + j], 1), :] = one
        for j in range(ROWS_PER_TRIP // 8):
            acc = acc + buf_ref[pl.ds(i * ROWS_PER_TRIP + j * 8, 8), :]
        return acc

    acc = jnp.zeros((8, D_FEAT), jnp.float32)
    for k in range(N_CHUNK):
        cps_in[k].wait()
        acc = jax.lax.fori_loop(k * TRIPS, (k + 1) * TRIPS, fused_body, acc)
    token = jnp.sum(acc, axis=0, keepdims=True) * (1.0 / N_NODES)

    cps_out = [
        pltpu.make_async_copy(
            buf_ref.at[pl.ds(k * OCH, OCH), :],
            out_ref.at[pl.ds(k * OCH, OCH), :],
            sem_out,
        )
        for k in range(N_OUT)
    ]
    for k in range(N_OUT):
        sl = pl.ds(k * OCH, OCH)
        m = mask_ref[sl, :]
        buf_ref[sl, :] = jnp.where(m > 0.0, token, buf_ref[sl, :])
        cps_out[k].start()
    for k in range(N_OUT):
        cps_out[k].wait()


def kernel(x, edge_index, mask_idx):
    idx = mask_idx.astype(jnp.int32)
    x_new = pl.pallas_call(
        _attrmask_body,
        out_shape=jax.ShapeDtypeStruct(x.shape, x.dtype),
        in_specs=[
            pl.BlockSpec(memory_space=pltpu.SMEM),
            pl.BlockSpec(memory_space=pltpu.HBM),
        ],
        out_specs=pl.BlockSpec(memory_space=pltpu.HBM),
        scratch_shapes=[
            pltpu.VMEM((N_NODES, D_FEAT), jnp.float32),
            pltpu.VMEM((N_NODES, 1), jnp.float32),
            pltpu.SemaphoreType.DMA,
            pltpu.SemaphoreType.DMA,
        ],
    )(idx, x)
    return (x_new, edge_index)


# speedup vs baseline: 8.9181x; 1.0302x over previous
import jax
import jax.numpy as jnp
from jax.experimental import pallas as pl
from jax.experimental.pallas import tpu as pltpu

N_NODES = 10000
D_FEAT = 128
MASK_NUM = 2000

N_CHUNK = 5
CHUNK = N_NODES // N_CHUNK
TRIPS = 25
ROWS_PER_TRIP = CHUNK // TRIPS
IDX_PER_TRIP = MASK_NUM // (N_CHUNK * TRIPS)
N_OUT = 8
OCH = N_NODES // N_OUT


def _attrmask_body(idx_ref, x_ref, out_ref, buf_ref, mask_ref, sem_in, sem_out):
    cps_in = [
        pltpu.make_async_copy(
            x_ref.at[pl.ds(k * CHUNK, CHUNK), :],
            buf_ref.at[pl.ds(k * CHUNK, CHUNK), :],
            sem_in,
        )
        for k in range(N_CHUNK)
    ]
    for cp in cps_in:
        cp.start()

    mask_ref[...] = jnp.zeros((N_NODES, 1), jnp.float32)
    one = jnp.ones((1, 1), jnp.float32)

    def fused_body(i, acc):
        for j in range(IDX_PER_TRIP):
            mask_ref[pl.ds(idx_ref[i * IDX_PER_TRIP + j], 1), :] = one
        for j in range(ROWS_PER_TRIP // 8):
            acc = acc + buf_ref[pl.ds(i * ROWS_PER_TRIP + j * 8, 8), :]
        return acc

    acc = jnp.zeros((8, D_FEAT), jnp.float32)
    for k in range(N_CHUNK):
        cps_in[k].wait()
        acc = jax.lax.fori_loop(
            k * TRIPS, (k + 1) * TRIPS, fused_body, acc, unroll=5
        )
    token = jnp.sum(acc, axis=0, keepdims=True) * (1.0 / N_NODES)

    cps_out = [
        pltpu.make_async_copy(
            buf_ref.at[pl.ds(k * OCH, OCH), :],
            out_ref.at[pl.ds(k * OCH, OCH), :],
            sem_out,
        )
        for k in range(N_OUT)
    ]
    for k in range(N_OUT):
        sl = pl.ds(k * OCH, OCH)
        m = mask_ref[sl, :]
        buf_ref[sl, :] = jnp.where(m > 0.0, token, buf_ref[sl, :])
        cps_out[k].start()
    for k in range(N_OUT):
        cps_out[k].wait()


def kernel(x, edge_index, mask_idx):
    idx = mask_idx.astype(jnp.int32)
    x_new = pl.pallas_call(
        _attrmask_body,
        out_shape=jax.ShapeDtypeStruct(x.shape, x.dtype),
        in_specs=[
            pl.BlockSpec(memory_space=pltpu.SMEM),
            pl.BlockSpec(memory_space=pltpu.HBM),
        ],
        out_specs=pl.BlockSpec(memory_space=pltpu.HBM),
        scratch_shapes=[
            pltpu.VMEM((N_NODES, D_FEAT), jnp.float32),
            pltpu.VMEM((N_NODES, 1), jnp.float32),
            pltpu.SemaphoreType.DMA,
            pltpu.SemaphoreType.DMA,
        ],
    )(idx, x)
    return (x_new, edge_index)


# speedup vs baseline: 9.3996x; 1.0540x over previous
import jax
import jax.numpy as jnp
from jax.experimental import pallas as pl
from jax.experimental.pallas import tpu as pltpu

N_NODES = 10000
D_FEAT = 128
MASK_NUM = 2000

N_CHUNK = 5
CHUNK = N_NODES // N_CHUNK
TRIPS = 25
ROWS_PER_TRIP = CHUNK // TRIPS
IDX_PER_TRIP = MASK_NUM // (N_CHUNK * TRIPS)
N_OUT = 8
OCH = N_NODES // N_OUT


def _attrmask_body(idx_ref, x_ref, out_ref, buf_ref, mask_ref, sem_in, sem_out):
    cps_in = [
        pltpu.make_async_copy(
            x_ref.at[pl.ds(k * CHUNK, CHUNK), :],
            buf_ref.at[pl.ds(k * CHUNK, CHUNK), :],
            sem_in,
        )
        for k in range(N_CHUNK)
    ]
    for cp in cps_in:
        cp.start()

    mask_ref[...] = jnp.zeros((N_NODES, 1), jnp.float32)
    one = jnp.ones((1, 1), jnp.float32)

    def fused_body(i, acc):
        for j in range(IDX_PER_TRIP):
            mask_ref[pl.ds(idx_ref[i * IDX_PER_TRIP + j], 1), :] = one
        for j in range(ROWS_PER_TRIP // 8):
            acc = acc + buf_ref[pl.ds(i * ROWS_PER_TRIP + j * 8, 8), :]
        return acc

    acc = jnp.zeros((8, D_FEAT), jnp.float32)
    for k in range(N_CHUNK):
        cps_in[k].wait()
        acc = jax.lax.fori_loop(
            k * TRIPS, (k + 1) * TRIPS, fused_body, acc, unroll=25
        )
    token = jnp.sum(acc, axis=0, keepdims=True) * (1.0 / N_NODES)

    cps_out = [
        pltpu.make_async_copy(
            buf_ref.at[pl.ds(k * OCH, OCH), :],
            out_ref.at[pl.ds(k * OCH, OCH), :],
            sem_out,
        )
        for k in range(N_OUT)
    ]
    for k in range(N_OUT):
        sl = pl.ds(k * OCH, OCH)
        m = mask_ref[sl, :]
        buf_ref[sl, :] = jnp.where(m > 0.0, token, buf_ref[sl, :])
        cps_out[k].start()
    for k in range(N_OUT):
        cps_out[k].wait()


def kernel(x, edge_index, mask_idx):
    idx = mask_idx.astype(jnp.int32)
    x_new = pl.pallas_call(
        _attrmask_body,
        out_shape=jax.ShapeDtypeStruct(x.shape, x.dtype),
        in_specs=[
            pl.BlockSpec(memory_space=pltpu.SMEM),
            pl.BlockSpec(memory_space=pltpu.HBM),
        ],
        out_specs=pl.BlockSpec(memory_space=pltpu.HBM),
        scratch_shapes=[
            pltpu.VMEM((N_NODES, D_FEAT), jnp.float32),
            pltpu.VMEM((N_NODES, 1), jnp.float32),
            pltpu.SemaphoreType.DMA,
            pltpu.SemaphoreType.DMA,
        ],
    )(idx, x)
    return (x_new, edge_index)


# speedup vs baseline: 9.4576x; 1.0062x over previous
import jax
import jax.numpy as jnp
from jax.experimental import pallas as pl
from jax.experimental.pallas import tpu as pltpu

N_NODES = 10000
D_FEAT = 128
MASK_NUM = 2000

N_CHUNK = 5
CHUNK = N_NODES // N_CHUNK
TRIPS = 25
ROWS_PER_TRIP = CHUNK // TRIPS
IDX_PER_TRIP = MASK_NUM // (N_CHUNK * TRIPS)
N_OUT = 16
OCH = N_NODES // N_OUT


def _attrmask_body(idx_ref, x_ref, out_ref, buf_ref, mask_ref, sem_in, sem_out):
    cps_in = [
        pltpu.make_async_copy(
            x_ref.at[pl.ds(k * CHUNK, CHUNK), :],
            buf_ref.at[pl.ds(k * CHUNK, CHUNK), :],
            sem_in,
        )
        for k in range(N_CHUNK)
    ]
    for cp in cps_in:
        cp.start()

    mask_ref[...] = jnp.zeros((N_NODES, 1), jnp.float32)
    one = jnp.ones((1, 1), jnp.float32)

    def fused_body(i, acc):
        for j in range(IDX_PER_TRIP):
            mask_ref[pl.ds(idx_ref[i * IDX_PER_TRIP + j], 1), :] = one
        for j in range(ROWS_PER_TRIP // 8):
            acc = acc + buf_ref[pl.ds(i * ROWS_PER_TRIP + j * 8, 8), :]
        return acc

    acc = jnp.zeros((8, D_FEAT), jnp.float32)
    for k in range(N_CHUNK):
        cps_in[k].wait()
        acc = jax.lax.fori_loop(
            k * TRIPS, (k + 1) * TRIPS, fused_body, acc, unroll=25
        )
    token = jnp.sum(acc, axis=0, keepdims=True) * (1.0 / N_NODES)

    cps_out = [
        pltpu.make_async_copy(
            buf_ref.at[pl.ds(k * OCH, OCH), :],
            out_ref.at[pl.ds(k * OCH, OCH), :],
            sem_out,
        )
        for k in range(N_OUT)
    ]
    for k in range(N_OUT):
        sl = pl.ds(k * OCH, OCH)
        m = mask_ref[sl, :]
        buf_ref[sl, :] = jnp.where(m > 0.0, token, buf_ref[sl, :])
        cps_out[k].start()
    for k in range(N_OUT):
        cps_out[k].wait()


def kernel(x, edge_index, mask_idx):
    idx = mask_idx.astype(jnp.int32)
    x_new = pl.pallas_call(
        _attrmask_body,
        out_shape=jax.ShapeDtypeStruct(x.shape, x.dtype),
        in_specs=[
            pl.BlockSpec(memory_space=pltpu.SMEM),
            pl.BlockSpec(memory_space=pltpu.HBM),
        ],
        out_specs=pl.BlockSpec(memory_space=pltpu.HBM),
        scratch_shapes=[
            pltpu.VMEM((N_NODES, D_FEAT), jnp.float32),
            pltpu.VMEM((N_NODES, 1), jnp.float32),
            pltpu.SemaphoreType.DMA,
            pltpu.SemaphoreType.DMA,
        ],
    )(idx, x)
    return (x_new, edge_index)


# speedup vs baseline: 9.5920x; 1.0142x over previous
import jax
import jax.numpy as jnp
from jax.experimental import pallas as pl
from jax.experimental.pallas import tpu as pltpu

N_NODES = 10000
D_FEAT = 128
MASK_NUM = 2000

N_CHUNK = 10
CHUNK = N_NODES // N_CHUNK
TRIPS = 25
ROWS_PER_TRIP = CHUNK // TRIPS
IDX_PER_TRIP = MASK_NUM // (N_CHUNK * TRIPS)
N_OUT = 16
OCH = N_NODES // N_OUT


def _attrmask_body(idx_ref, x_ref, out_ref, buf_ref, mask_ref, sem_in, sem_out):
    cps_in = [
        pltpu.make_async_copy(
            x_ref.at[pl.ds(k * CHUNK, CHUNK), :],
            buf_ref.at[pl.ds(k * CHUNK, CHUNK), :],
            sem_in,
        )
        for k in range(N_CHUNK)
    ]
    for cp in cps_in:
        cp.start()

    mask_ref[...] = jnp.zeros((N_NODES, 1), jnp.float32)
    one = jnp.ones((1, 1), jnp.float32)

    def fused_body(i, acc):
        for j in range(IDX_PER_TRIP):
            mask_ref[pl.ds(idx_ref[i * IDX_PER_TRIP + j], 1), :] = one
        for j in range(ROWS_PER_TRIP // 8):
            acc = acc + buf_ref[pl.ds(i * ROWS_PER_TRIP + j * 8, 8), :]
        return acc

    acc = jnp.zeros((8, D_FEAT), jnp.float32)
    for k in range(N_CHUNK):
        cps_in[k].wait()
        acc = jax.lax.fori_loop(
            k * TRIPS, (k + 1) * TRIPS, fused_body, acc, unroll=25
        )
    token = jnp.sum(acc, axis=0, keepdims=True) * (1.0 / N_NODES)

    cps_out = [
        pltpu.make_async_copy(
            buf_ref.at[pl.ds(k * OCH, OCH), :],
            out_ref.at[pl.ds(k * OCH, OCH), :],
            sem_out,
        )
        for k in range(N_OUT)
    ]
    for k in range(N_OUT):
        sl = pl.ds(k * OCH, OCH)
        m = mask_ref[sl, :]
        buf_ref[sl, :] = jnp.where(m > 0.0, token, buf_ref[sl, :])
        cps_out[k].start()
    for k in range(N_OUT):
        cps_out[k].wait()


def kernel(x, edge_index, mask_idx):
    idx = mask_idx.astype(jnp.int32)
    x_new = pl.pallas_call(
        _attrmask_body,
        out_shape=jax.ShapeDtypeStruct(x.shape, x.dtype),
        in_specs=[
            pl.BlockSpec(memory_space=pltpu.SMEM),
            pl.BlockSpec(memory_space=pltpu.HBM),
        ],
        out_specs=pl.BlockSpec(memory_space=pltpu.HBM),
        scratch_shapes=[
            pltpu.VMEM((N_NODES, D_FEAT), jnp.float32),
            pltpu.VMEM((N_NODES, 1), jnp.float32),
            pltpu.SemaphoreType.DMA,
            pltpu.SemaphoreType.DMA,
        ],
    )(idx, x)
    return (x_new, edge_index)


# speedup vs baseline: 9.7071x; 1.0120x over previous
import jax
import jax.numpy as jnp
from jax.experimental import pallas as pl
from jax.experimental.pallas import tpu as pltpu

N_NODES = 10000
D_FEAT = 128
MASK_NUM = 2000

N_CHUNK = 25
CHUNK = N_NODES // N_CHUNK
TRIPS = 10
ROWS_PER_TRIP = CHUNK // TRIPS
IDX_PER_TRIP = MASK_NUM // (N_CHUNK * TRIPS)
N_OUT = 16
OCH = N_NODES // N_OUT


def _attrmask_body(idx_ref, x_ref, out_ref, buf_ref, mask_ref, sem_in, sem_out):
    cps_in = [
        pltpu.make_async_copy(
            x_ref.at[pl.ds(k * CHUNK, CHUNK), :],
            buf_ref.at[pl.ds(k * CHUNK, CHUNK), :],
            sem_in,
        )
        for k in range(N_CHUNK)
    ]
    for cp in cps_in:
        cp.start()

    mask_ref[...] = jnp.zeros((N_NODES, 1), jnp.float32)
    one = jnp.ones((1, 1), jnp.float32)

    def fused_body(i, acc):
        for j in range(IDX_PER_TRIP):
            mask_ref[pl.ds(idx_ref[i * IDX_PER_TRIP + j], 1), :] = one
        for j in range(ROWS_PER_TRIP // 8):
            acc = acc + buf_ref[pl.ds(i * ROWS_PER_TRIP + j * 8, 8), :]
        return acc

    acc = jnp.zeros((8, D_FEAT), jnp.float32)
    for k in range(N_CHUNK):
        cps_in[k].wait()
        acc = jax.lax.fori_loop(
            k * TRIPS, (k + 1) * TRIPS, fused_body, acc, unroll=TRIPS
        )
    token = jnp.sum(acc, axis=0, keepdims=True) * (1.0 / N_NODES)

    cps_out = [
        pltpu.make_async_copy(
            buf_ref.at[pl.ds(k * OCH, OCH), :],
            out_ref.at[pl.ds(k * OCH, OCH), :],
            sem_out,
        )
        for k in range(N_OUT)
    ]
    for k in range(N_OUT):
        sl = pl.ds(k * OCH, OCH)
        m = mask_ref[sl, :]
        buf_ref[sl, :] = jnp.where(m > 0.0, token, buf_ref[sl, :])
        cps_out[k].start()
    for k in range(N_OUT):
        cps_out[k].wait()


def kernel(x, edge_index, mask_idx):
    idx = mask_idx.astype(jnp.int32)
    x_new = pl.pallas_call(
        _attrmask_body,
        out_shape=jax.ShapeDtypeStruct(x.shape, x.dtype),
        in_specs=[
            pl.BlockSpec(memory_space=pltpu.SMEM),
            pl.BlockSpec(memory_space=pltpu.HBM),
        ],
        out_specs=pl.BlockSpec(memory_space=pltpu.HBM),
        scratch_shapes=[
            pltpu.VMEM((N_NODES, D_FEAT), jnp.float32),
            pltpu.VMEM((N_NODES, 1), jnp.float32),
            pltpu.SemaphoreType.DMA,
            pltpu.SemaphoreType.DMA,
        ],
    )(idx, x)
    return (x_new, edge_index)


# speedup vs baseline: 9.7585x; 1.0053x over previous
import jax
import jax.numpy as jnp
from jax.experimental import pallas as pl
from jax.experimental.pallas import tpu as pltpu

N_NODES = 10000
D_FEAT = 128
MASK_NUM = 2000

N_CHUNK = 25
CHUNK = N_NODES // N_CHUNK
TRIPS = 10
ROWS_PER_TRIP = CHUNK // TRIPS
IDX_PER_TRIP = MASK_NUM // (N_CHUNK * TRIPS)
N_OUT = 25
OCH = N_NODES // N_OUT


def _attrmask_body(idx_ref, x_ref, out_ref, buf_ref, mask_ref, sem_in, sem_out):
    cps_in = [
        pltpu.make_async_copy(
            x_ref.at[pl.ds(k * CHUNK, CHUNK), :],
            buf_ref.at[pl.ds(k * CHUNK, CHUNK), :],
            sem_in,
        )
        for k in range(N_CHUNK)
    ]
    for cp in cps_in:
        cp.start()

    mask_ref[...] = jnp.zeros((N_NODES, 1), jnp.float32)
    one = jnp.ones((1, 1), jnp.float32)

    def fused_body(i, acc):
        for j in range(IDX_PER_TRIP):
            mask_ref[pl.ds(idx_ref[i * IDX_PER_TRIP + j], 1), :] = one
        for j in range(ROWS_PER_TRIP // 8):
            acc = acc + buf_ref[pl.ds(i * ROWS_PER_TRIP + j * 8, 8), :]
        return acc

    acc = jnp.zeros((8, D_FEAT), jnp.float32)
    for k in range(N_CHUNK):
        cps_in[k].wait()
        acc = jax.lax.fori_loop(
            k * TRIPS, (k + 1) * TRIPS, fused_body, acc, unroll=TRIPS
        )
    token = jnp.sum(acc, axis=0, keepdims=True) * (1.0 / N_NODES)

    cps_out = [
        pltpu.make_async_copy(
            buf_ref.at[pl.ds(k * OCH, OCH), :],
            out_ref.at[pl.ds(k * OCH, OCH), :],
            sem_out,
        )
        for k in range(N_OUT)
    ]
    for k in range(N_OUT):
        sl = pl.ds(k * OCH, OCH)
        m = mask_ref[sl, :]
        buf_ref[sl, :] = jnp.where(m > 0.0, token, buf_ref[sl, :])
        cps_out[k].start()
    for k in range(N_OUT):
        cps_out[k].wait()


def kernel(x, edge_index, mask_idx):
    idx = mask_idx.astype(jnp.int32)
    x_new = pl.pallas_call(
        _attrmask_body,
        out_shape=jax.ShapeDtypeStruct(x.shape, x.dtype),
        in_specs=[
            pl.BlockSpec(memory_space=pltpu.SMEM),
            pl.BlockSpec(memory_space=pltpu.HBM),
        ],
        out_specs=pl.BlockSpec(memory_space=pltpu.HBM),
        scratch_shapes=[
            pltpu.VMEM((N_NODES, D_FEAT), jnp.float32),
            pltpu.VMEM((N_NODES, 1), jnp.float32),
            pltpu.SemaphoreType.DMA,
            pltpu.SemaphoreType.DMA,
        ],
    )(idx, x)
    return (x_new, edge_index)


# speedup vs baseline: 9.7603x; 1.0002x over previous
import jax
import jax.numpy as jnp
from jax.experimental import pallas as pl
from jax.experimental.pallas import tpu as pltpu

N_NODES = 10000
D_FEAT = 128
MASK_NUM = 2000

N_CHUNK = 50
CHUNK = N_NODES // N_CHUNK
TRIPS = 5
ROWS_PER_TRIP = CHUNK // TRIPS
IDX_PER_TRIP = MASK_NUM // (N_CHUNK * TRIPS)
N_OUT = 25
OCH = N_NODES // N_OUT


def _attrmask_body(idx_ref, x_ref, out_ref, buf_ref, mask_ref, sem_in, sem_out):
    cps_in = [
        pltpu.make_async_copy(
            x_ref.at[pl.ds(k * CHUNK, CHUNK), :],
            buf_ref.at[pl.ds(k * CHUNK, CHUNK), :],
            sem_in,
        )
        for k in range(N_CHUNK)
    ]
    for cp in cps_in:
        cp.start()

    mask_ref[...] = jnp.zeros((N_NODES, 1), jnp.float32)
    one = jnp.ones((1, 1), jnp.float32)

    def fused_body(i, acc):
        for j in range(IDX_PER_TRIP):
            mask_ref[pl.ds(idx_ref[i * IDX_PER_TRIP + j], 1), :] = one
        for j in range(ROWS_PER_TRIP // 8):
            acc = acc + buf_ref[pl.ds(i * ROWS_PER_TRIP + j * 8, 8), :]
        return acc

    acc = jnp.zeros((8, D_FEAT), jnp.float32)
    for k in range(N_CHUNK):
        cps_in[k].wait()
        acc = jax.lax.fori_loop(
            k * TRIPS, (k + 1) * TRIPS, fused_body, acc, unroll=TRIPS
        )
    token = jnp.sum(acc, axis=0, keepdims=True) * (1.0 / N_NODES)

    cps_out = [
        pltpu.make_async_copy(
            buf_ref.at[pl.ds(k * OCH, OCH), :],
            out_ref.at[pl.ds(k * OCH, OCH), :],
            sem_out,
        )
        for k in range(N_OUT)
    ]
    for k in range(N_OUT):
        sl = pl.ds(k * OCH, OCH)
        m = mask_ref[sl, :]
        buf_ref[sl, :] = jnp.where(m > 0.0, token, buf_ref[sl, :])
        cps_out[k].start()
    for k in range(N_OUT):
        cps_out[k].wait()


def kernel(x, edge_index, mask_idx):
    idx = mask_idx.astype(jnp.int32)
    x_new = pl.pallas_call(
        _attrmask_body,
        out_shape=jax.ShapeDtypeStruct(x.shape, x.dtype),
        in_specs=[
            pl.BlockSpec(memory_space=pltpu.SMEM),
            pl.BlockSpec(memory_space=pltpu.HBM),
        ],
        out_specs=pl.BlockSpec(memory_space=pltpu.HBM),
        scratch_shapes=[
            pltpu.VMEM((N_NODES, D_FEAT), jnp.float32),
            pltpu.VMEM((N_NODES, 1), jnp.float32),
            pltpu.SemaphoreType.DMA,
            pltpu.SemaphoreType.DMA,
        ],
    )(idx, x)
    return (x_new, edge_index)
